# baseline (speedup 1.0000x reference)
import functools
import math

import jax
import jax.numpy as jnp
from jax import lax
from jax.experimental import pallas as pl
from jax.experimental.pallas import tpu as pltpu
from jax.experimental.pallas import tpu_sc as plsc

_NC = 2
_NS = 16
_NW = _NC * _NS
_LANES = 16

_NBUF = 4
_LA = 3


@functools.lru_cache(maxsize=None)
def _build(n_seq, seq_len, vocab, d_model, scale):
    assert n_seq % (_NW * 128) == 0 and vocab % 2 == 0
    bw = 128
    d2 = 2 * d_model
    assert d2 == 128
    groups = bw // _LANES
    assert seq_len % _NBUF == 0

    mesh = plsc.VectorSubcoreMesh(core_axis_name="c", subcore_axis_name="s")

    @functools.partial(
        pl.kernel,
        mesh=mesh,
        out_type=jax.ShapeDtypeStruct((seq_len, d_model, n_seq), jnp.float32),
        scratch_types=[
            pltpu.VMEM((seq_len, bw), jnp.int32),
            [pltpu.VMEM((bw,), jnp.int32) for _ in range(_NBUF)],
            [pltpu.VMEM((bw, d2), jnp.float32) for _ in range(_NBUF)],
            [pltpu.VMEM((d_model, bw), jnp.float32) for _ in range(_NBUF)],
            [pltpu.SemaphoreType.DMA for _ in range(_NBUF)],
            [pltpu.SemaphoreType.DMA for _ in range(_NBUF)],
        ],
        compiler_params=pltpu.CompilerParams(needs_layout_passes=False),
    )
    def emb(xt_hbm, t2_hbm, out_hbm, idx_v, pidx_v, grows_v, tbuf_v,
            gsems, osems):
        wid = lax.axis_index("s") * _NC + lax.axis_index("c")
        b0 = wid * bw

        pltpu.sync_copy(xt_hbm.at[:, pl.ds(b0, bw)], idx_v)

        lanes = lax.iota(jnp.int32, _LANES)

        def gdesc(s, b):
            return pltpu.make_async_copy(
                t2_hbm.at[pidx_v[b]], grows_v[b], gsems[b]
            )

        def wdesc(s, b):
            return pltpu.make_async_copy(
                tbuf_v[b], out_hbm.at[s, :, pl.ds(b0, bw)], osems[b]
            )

        def fire_gather(s, b):
            for g in range(groups):
                sl = pl.ds(g * _LANES, _LANES)
                pidx_v[b][sl] = idx_v[s, sl] >> 1
            gdesc(s, b).start()

        def transform(s, b):
            gdesc(s, b).wait()
            for g in range(groups):
                sl = pl.ds(g * _LANES, _LANES)
                rows = lanes + (g * _LANES)
                hcol = (idx_v[s, sl] & 1) * d_model

                @plsc.parallel_loop(0, d_model, 1, unroll=8)
                def _(d):
                    v = plsc.load_gather(grows_v[b], [rows, hcol + d])
                    tbuf_v[b][d, sl] = v * scale

            wdesc(s, b).start()

        for s in range(_LA):
            fire_gather(s, s % _NBUF)

        def outer(so, carry):
            s0 = so * _NBUF
            for b in range(_NBUF):
                s = s0 + b

                @pl.when(s + _LA < seq_len)
                def _():
                    bla = (b + _LA) % _NBUF

                    @pl.when(s + _LA >= _NBUF)
                    def _():
                        wdesc(0, bla).wait()
                    fire_gather(s + _LA, bla)

                transform(s, b)
            return carry

        lax.fori_loop(0, seq_len // _NBUF, outer, 0)

        for j in range(seq_len - _NBUF, seq_len):
            wdesc(j, j % _NBUF).wait()

    return emb


def kernel(x, table):
    vocab, d_model = table.shape
    n_seq, seq_len = x.shape
    scale = float(math.sqrt(d_model))
    xt = x.T.astype(jnp.int32)
    t2 = table.reshape(vocab // 2, 2 * d_model)
    out = _build(n_seq, seq_len, vocab, d_model, scale)(xt, t2)
    return jnp.transpose(out, (2, 0, 1))


# speedup vs baseline: 1.0822x; 1.0822x over previous
import functools
import math

import jax
import jax.numpy as jnp
from jax import lax
from jax.experimental import pallas as pl
from jax.experimental.pallas import tpu as pltpu
from jax.experimental.pallas import tpu_sc as plsc

_NC = 2
_NS = 16
_NW = _NC * _NS
_LANES = 16

_NBUF = 4


@functools.lru_cache(maxsize=None)
def _build(n_seq, seq_len, vocab, d_model, scale):
    rows_per_w = n_seq // _NW
    assert n_seq % _NW == 0
    c0 = min(128, seq_len - seq_len % 8 if seq_len <= 128 else 128)
    lens = (c0, seq_len - c0)
    offs = (0, c0)
    assert all(l % 8 == 0 and 0 < l <= 128 for l in lens)
    n_chunks = rows_per_w * 2
    assert n_chunks % _NBUF == 0
    d_regs = d_model // _LANES

    mesh = plsc.VectorSubcoreMesh(core_axis_name="c", subcore_axis_name="s")

    @functools.partial(
        pl.kernel,
        mesh=mesh,
        out_type=jax.ShapeDtypeStruct((n_seq, seq_len, d_model), jnp.float32),
        scratch_types=[
            pltpu.VMEM((rows_per_w, seq_len), jnp.int32),
            [pltpu.VMEM((lens[b % 2], d_model), jnp.float32)
             for b in range(_NBUF)],
            [pltpu.SemaphoreType.DMA for _ in range(_NBUF)],
            [pltpu.SemaphoreType.DMA for _ in range(_NBUF)],
        ],
        compiler_params=pltpu.CompilerParams(use_tc_tiling_on_sc=False),
    )
    def emb(x_hbm, table_hbm, out_hbm, idx_v, rows_v, gsems, osems):
        wid = lax.axis_index("s") * _NC + lax.axis_index("c")
        base = wid * rows_per_w

        pltpu.sync_copy(x_hbm.at[pl.ds(base, rows_per_w)], idx_v)

        def gdesc(g, b):
            r = g // 2
            return pltpu.make_async_copy(
                table_hbm.at[idx_v.at[r, pl.ds(offs[b % 2], lens[b % 2])]],
                rows_v[b],
                gsems[b],
            )

        def wdesc(g, b):
            r = g // 2
            return pltpu.make_async_copy(
                rows_v[b],
                out_hbm.at[base + r, pl.ds(offs[b % 2], lens[b % 2])],
                osems[b],
            )

        def scale_and_emit(g, b):
            gdesc(g, b).wait()

            @plsc.parallel_loop(0, lens[b % 2], 1, unroll=8)
            def _(i):
                for j in range(d_regs):
                    sl = pl.ds(j * _LANES, _LANES)
                    rows_v[b][i, sl] = rows_v[b][i, sl] * scale

            wdesc(g, b).start()

        for b in range(_NBUF):
            gdesc(b, b).start()

        def outer(go, carry):
            g0 = go * _NBUF
            for b in range(_NBUF):
                scale_and_emit(g0 + b, b)
            @pl.when(g0 + _NBUF < n_chunks)
            def _():
                for b in range(_NBUF):
                    wdesc(g0 + b, b).wait()
                    gdesc(g0 + _NBUF + b, b).start()

            return carry

        lax.fori_loop(0, n_chunks // _NBUF, outer, 0)

        for b in range(_NBUF):
            wdesc(n_chunks - _NBUF + b, b).wait()

    return emb


def kernel(x, table):
    vocab, d_model = table.shape
    n_seq, seq_len = x.shape
    scale = float(math.sqrt(d_model))
    xi = x.astype(jnp.int32)
    return _build(n_seq, seq_len, vocab, d_model, scale)(xi, table)
